# baseline (speedup 1.0000x reference)
import jax
import jax.numpy as jnp
from jax.experimental import pallas as pl
from jax.experimental.pallas import tpu as pltpu

_C = 512
_NBUF = 4


def _body(x_hbm, p_hbm, o_hbm, buf, pos_v, in_sems, out_sems, p_sem):
    N = x_hbm.shape[0]
    T = p_hbm.shape[0]
    n_chunks = N // _C

    pltpu.make_async_copy(p_hbm, pos_v, p_sem).start()
    for i in range(min(_NBUF, n_chunks)):
        pltpu.make_async_copy(
            x_hbm.at[pl.ds(i * _C, _C)], buf.at[i], in_sems.at[i]
        ).start()
    pltpu.make_async_copy(p_hbm, pos_v, p_sem).wait()

    for i in range(n_chunks):
        slot = i % _NBUF
        pltpu.make_async_copy(
            x_hbm.at[pl.ds(i * _C, _C)], buf.at[slot], in_sems.at[slot]
        ).wait()
        prow = (i * _C) % T
        buf[slot] = buf[slot] + pos_v[pl.ds(prow, _C)]
        pltpu.make_async_copy(
            buf.at[slot], o_hbm.at[pl.ds(i * _C, _C)], out_sems.at[slot]
        ).start()
        nxt = i + _NBUF
        if nxt < n_chunks:
            pltpu.make_async_copy(
                buf.at[slot], o_hbm.at[pl.ds(i * _C, _C)], out_sems.at[slot]
            ).wait()
            pltpu.make_async_copy(
                x_hbm.at[pl.ds(nxt * _C, _C)], buf.at[slot], in_sems.at[slot]
            ).start()

    for i in range(max(n_chunks - _NBUF, 0), n_chunks):
        slot = i % _NBUF
        pltpu.make_async_copy(
            buf.at[slot], o_hbm.at[pl.ds(i * _C, _C)], out_sems.at[slot]
        ).wait()


def kernel(x, pos_table):
    T, D = pos_table.shape
    xf = x.reshape(-1, D)
    N = xf.shape[0]
    out = pl.pallas_call(
        _body,
        in_specs=[
            pl.BlockSpec(memory_space=pltpu.MemorySpace.HBM),
            pl.BlockSpec(memory_space=pltpu.MemorySpace.HBM),
        ],
        out_specs=pl.BlockSpec(memory_space=pltpu.MemorySpace.HBM),
        out_shape=jax.ShapeDtypeStruct((N, D), x.dtype),
        scratch_shapes=[
            pltpu.VMEM((_NBUF, _C, D), jnp.float32),
            pltpu.VMEM((T, D), jnp.float32),
            pltpu.SemaphoreType.DMA((_NBUF,)),
            pltpu.SemaphoreType.DMA((_NBUF,)),
            pltpu.SemaphoreType.DMA,
        ],
    )(xf, pos_table)
    return out.reshape(-1, T, D)


# speedup vs baseline: 1.4742x; 1.4742x over previous
import jax
import jax.numpy as jnp
from jax.experimental import pallas as pl


def _add_body(x_ref, p_ref, o_ref):
    T = p_ref.shape[0]
    o_ref[:T] = x_ref[:T] + p_ref[...]
    o_ref[T:] = x_ref[T:] + p_ref[...]


def kernel(x, pos_table):
    T, D = pos_table.shape
    xf = x.reshape(-1, D)
    N = xf.shape[0]
    BR = 2 * T
    grid = (N // BR,)
    out = pl.pallas_call(
        _add_body,
        grid=grid,
        in_specs=[
            pl.BlockSpec((BR, D), lambda i: (i, 0)),
            pl.BlockSpec((T, D), lambda i: (0, 0)),
        ],
        out_specs=pl.BlockSpec((BR, D), lambda i: (i, 0)),
        out_shape=jax.ShapeDtypeStruct((N, D), x.dtype),
    )(xf, pos_table)
    return out.reshape(-1, T, D)
